# baseline (speedup 1.0000x reference)
import jax
import jax.numpy as jnp
from jax.experimental import pallas as pl
from jax.experimental.pallas import tpu as pltpu

_W = 1024
_CH = 512
_NROWS = 32 * 1024
_NSTEPS = _NROWS // _CH


def _scan_chunk(x_slot, o_slot):
    v = x_slot[...]
    s = 1
    while s < _W:
        v = jnp.maximum(v, jnp.pad(v[:, s:], ((0, 0), (0, s)),
                                   constant_values=-jnp.inf))
        s *= 2
    o_slot[...] = v


def _pipeline_body(x_hbm, o_hbm, x_buf, o_buf, in_sem, out_sem):
    def dma_in(slot, step):
        pltpu.make_async_copy(x_hbm.at[pl.ds(step * _CH, _CH)],
                              x_buf.at[slot], in_sem.at[slot]).start()

    def wait_in(slot):
        pltpu.make_async_copy(x_hbm.at[pl.ds(0, _CH)],
                              x_buf.at[slot], in_sem.at[slot]).wait()

    def dma_out(slot, step):
        pltpu.make_async_copy(o_buf.at[slot],
                              o_hbm.at[pl.ds(step * _CH, _CH)],
                              out_sem.at[slot]).start()

    def wait_out(slot):
        pltpu.make_async_copy(o_buf.at[slot],
                              o_hbm.at[pl.ds(0, _CH)],
                              out_sem.at[slot]).wait()

    dma_in(0, 0)

    def body(step, _):
        cur = jax.lax.rem(step, 2)
        nxt = jax.lax.rem(step + 1, 2)

        @pl.when(step + 1 < _NSTEPS)
        def _():
            dma_in(nxt, step + 1)

        wait_in(cur)

        @pl.when(step >= 2)
        def _():
            wait_out(cur)

        _scan_chunk(x_buf.at[cur], o_buf.at[cur])
        dma_out(cur, step)
        return ()

    jax.lax.fori_loop(0, _NSTEPS, body, ())
    wait_out(jax.lax.rem(_NSTEPS - 2, 2))
    wait_out(jax.lax.rem(_NSTEPS - 1, 2))


@jax.jit
def kernel(x):
    b, c, h, w = x.shape
    flat = x.reshape(b * c * h, w)
    out = pl.pallas_call(
        _pipeline_body,
        in_specs=[pl.BlockSpec(memory_space=pl.ANY)],
        out_specs=pl.BlockSpec(memory_space=pl.ANY),
        out_shape=jax.ShapeDtypeStruct(flat.shape, flat.dtype),
        scratch_shapes=[
            pltpu.VMEM((2, _CH, w), jnp.float32),
            pltpu.VMEM((2, _CH, w), jnp.float32),
            pltpu.SemaphoreType.DMA((2,)),
            pltpu.SemaphoreType.DMA((2,)),
        ],
    )(flat)
    return out.reshape(b, c, h, w)


# speedup vs baseline: 1.0003x; 1.0003x over previous
import jax
import jax.numpy as jnp
from jax.experimental import pallas as pl
from jax.experimental.pallas import tpu as pltpu

_W = 1024
_BR = 2048


def _revcummax_body(x_ref, o_ref):
    v = x_ref[...]
    s = 1
    while s < _W:
        shifted = jnp.pad(v[:, s:], ((0, 0), (0, s)),
                          constant_values=-jnp.inf)
        v = jnp.maximum(v, shifted)
        s *= 2
    o_ref[...] = v


@jax.jit
def kernel(x):
    b, c, h, w = x.shape
    flat = x.reshape(b * c * h, w)
    out = pl.pallas_call(
        _revcummax_body,
        grid=(flat.shape[0] // _BR,),
        in_specs=[pl.BlockSpec((_BR, w), lambda i: (i, 0))],
        out_specs=pl.BlockSpec((_BR, w), lambda i: (i, 0)),
        out_shape=jax.ShapeDtypeStruct(flat.shape, flat.dtype),
        compiler_params=pltpu.CompilerParams(
            dimension_semantics=("parallel",)),
    )(flat)
    return out.reshape(b, c, h, w)


# speedup vs baseline: 1.0043x; 1.0040x over previous
import jax
import jax.numpy as jnp
from jax.experimental import pallas as pl
from jax.experimental.pallas import tpu as pltpu

_W = 1024
_BR = 1024


def _revcummax_body(x_ref, o_ref):
    v = x_ref[...]
    s = 1
    while s < _W:
        shifted = jnp.pad(v[:, s:], ((0, 0), (0, s)),
                          constant_values=-jnp.inf)
        v = jnp.maximum(v, shifted)
        s *= 2
    o_ref[...] = v


@jax.jit
def kernel(x):
    b, c, h, w = x.shape
    flat = x.reshape(b * c * h, w)
    out = pl.pallas_call(
        _revcummax_body,
        grid=(flat.shape[0] // _BR,),
        in_specs=[pl.BlockSpec((_BR, w), lambda i: (i, 0))],
        out_specs=pl.BlockSpec((_BR, w), lambda i: (i, 0)),
        out_shape=jax.ShapeDtypeStruct(flat.shape, flat.dtype),
        compiler_params=pltpu.CompilerParams(
            dimension_semantics=("parallel",)),
    )(flat)
    return out.reshape(b, c, h, w)
